# baseline (speedup 1.0000x reference)
import dataclasses
import functools

import jax
import jax.numpy as jnp
from jax import lax
from jax.experimental import pallas as pl
from jax.experimental.pallas import tpu as pltpu
from jax.experimental.pallas import tpu_sc as plsc

N = 10000
D = 256
MSG = 256
T = 4
E_PER = 40000

NACC = 10112
E_PAD = 40960
CH = 80
CHUNKS = E_PAD // (16 * CH)
HALF = 128
NB = 2000
PB = 250
PROWS = 2 * T * N // 32


def _sc_params():
    cp = pltpu.CompilerParams()
    fields = pltpu.CompilerParams.__dataclass_fields__
    if "needs_layout_passes" in fields:
        cp = dataclasses.replace(cp, needs_layout_passes=False)
    if "use_tc_tiling_on_sc" in fields:
        cp = dataclasses.replace(cp, use_tc_tiling_on_sc=False)
    return cp


def _tables(x, W, b8):
    nblk = N // NB

    def mm(x_ref, w_ref, b_ref, a_ref, t_ref):
        xa = x_ref[...]
        w = w_ref[0]
        a_ref[0, 0] = jnp.dot(xa, w[:D, :], preferred_element_type=jnp.float32)
        t_ref[0, 0] = (jnp.dot(xa, w[D:, :], preferred_element_type=jnp.float32)
                       + b_ref[0, 0][None, :])

    return pl.pallas_call(
        mm,
        grid=(nblk, 2, T),
        in_specs=[
            pl.BlockSpec((NB, D), lambda nb, c, t: (nb, 0)),
            pl.BlockSpec((1, 2 * D, HALF), lambda nb, c, t: (t, 0, c)),
            pl.BlockSpec((1, 8, HALF), lambda nb, c, t: (t, 0, c)),
        ],
        out_specs=[
            pl.BlockSpec((1, 1, NB, HALF), lambda nb, c, t: (c, t, nb, 0)),
            pl.BlockSpec((1, 1, NB, HALF), lambda nb, c, t: (c, t, nb, 0)),
        ],
        out_shape=[jax.ShapeDtypeStruct((2, T, N, HALF), jnp.float32)] * 2,
    )(x, W, b8)


def _pack_tables(tabAf, tabBf):
    mesh = plsc.VectorSubcoreMesh(core_axis_name="c", subcore_axis_name="s")
    rows = 2 * T * N

    @functools.partial(
        pl.kernel,
        out_type=[jax.ShapeDtypeStruct((rows, HALF // 2), jnp.int32)] * 2,
        mesh=mesh,
        compiler_params=_sc_params(),
        scratch_types=[
            pltpu.VMEM((PB, HALF), jnp.float32),
            pltpu.VMEM((PB, HALF // 2), jnp.int32),
        ],
    )
    def pack_kernel(a_hbm, b_hbm, oa_hbm, ob_hbm, fin, iout):
        c = lax.axis_index("c")
        s = lax.axis_index("s")
        w = s * 2 + c
        base = w * PROWS

        for src_hbm, dst_hbm in ((a_hbm, oa_hbm), (b_hbm, ob_hbm)):
            @pl.loop(0, PROWS // PB)
            def _(k):
                r0 = base + k * PB
                pltpu.sync_copy(src_hbm.at[pl.ds(r0, PB)], fin)

                @pl.loop(0, PB)
                def _(i):
                    for g in range(HALF // 32):
                        lo = fin[i, pl.ds(g * 32, 16)]
                        hi = fin[i, pl.ds(g * 32 + 16, 16)]
                        pk = plsc.pack(lo, hi,
                                       format=plsc.PackFormat.INTERLEAVED)
                        iout[i, pl.ds(g * 16, 16)] = plsc.bitcast(
                            pk, jnp.int32)

                pltpu.sync_copy(iout, dst_hbm.at[pl.ds(r0, PB)])

    return pack_kernel(tabAf, tabBf)


def _edge_stage(tabAf, tabBf, srcg, tgtg, tgtp):
    mesh = plsc.VectorSubcoreMesh(core_axis_name="c", subcore_axis_name="s")
    Q = T * CHUNKS

    @functools.partial(
        pl.kernel,
        out_type=jax.ShapeDtypeStruct((N, 2 * HALF), jnp.float32),
        mesh=mesh,
        compiler_params=_sc_params(),
        scratch_types=[
            pltpu.VMEM((4, 3, CH), jnp.int32),
            pltpu.VMEM((CH, HALF // 2), jnp.int32),
            pltpu.VMEM((CH, HALF // 2), jnp.int32),
            pltpu.VMEM((CH, HALF // 2), jnp.int32),
            pltpu.VMEM((CH, HALF // 2), jnp.int32),
            pltpu.VMEM((CH, HALF), jnp.float32),
            pltpu.VMEM((CH, HALF), jnp.float32),
            pltpu.VMEM_SHARED((NACC, HALF), jnp.float32),
            pltpu.SemaphoreType.DMA,
            pltpu.SemaphoreType.DMA,
            pltpu.SemaphoreType.DMA,
            pltpu.SemaphoreType.DMA,
            pltpu.SemaphoreType.DMA,
            pltpu.SemaphoreType.DMA,
            pltpu.SemaphoreType.DMA,
            pltpu.SemaphoreType.DMA,
            pltpu.SemaphoreType.DMA,
            pltpu.SemaphoreType.DMA,
        ],
    )
    def edge_kernel(tabA_hbm, tabB_hbm, srcg_hbm, tgtg_hbm, tgtp_hbm, out_hbm,
                    idxb, sb0, sb1, tb0, tb1, mb0, mb1, acc,
                    semI0, semI1, semI2, semI3,
                    semA0, semA1, semB0, semB1, semS0, semS1):
        c = lax.axis_index("c")
        s = lax.axis_index("s")
        sbuf, tbuf, mbuf = (sb0, sb1), (tb0, tb1), (mb0, mb1)
        semI = (semI0, semI1, semI2, semI3)
        semA, semB, semS = (semA0, semA1), (semB0, semB1), (semS0, semS1)
        tabAc = tabA_hbm.at[c]
        tabBc = tabB_hbm.at[c]

        @pl.loop(0, CH)
        def _(i):
            for j in range(HALF // 16):
                mb0[i, pl.ds(j * 16, 16)] = jnp.zeros((16,), jnp.float32)

        rows_per_sub = NACC // 16
        nz, rz = rows_per_sub // CH, rows_per_sub % CH

        @pl.loop(0, nz)
        def _(k):
            pltpu.sync_copy(mb0, acc.at[pl.ds(s * rows_per_sub + k * CH, CH)])

        if rz:
            pltpu.sync_copy(mb0.at[pl.ds(0, rz)],
                            acc.at[pl.ds(s * rows_per_sub + nz * CH, rz)])

        plsc.subcore_barrier()

        def _off(q):
            return ((q >> 5) * E_PAD + s * (CHUNKS * CH) + (q & 31) * CH)

        def issue_i(q, islot):
            off = _off(q)
            pltpu.async_copy(srcg_hbm.at[pl.ds(off, CH)], idxb.at[islot, 0],
                             semI[islot])
            pltpu.async_copy(tgtg_hbm.at[pl.ds(off, CH)], idxb.at[islot, 1],
                             semI[islot])
            pltpu.async_copy(tgtp_hbm.at[pl.ds(off, CH)], idxb.at[islot, 2],
                             semI[islot])

        def wait_i(q, islot):
            off = _off(q)
            pltpu.make_async_copy(srcg_hbm.at[pl.ds(off, CH)],
                                  idxb.at[islot, 0], semI[islot]).wait()
            pltpu.make_async_copy(tgtg_hbm.at[pl.ds(off, CH)],
                                  idxb.at[islot, 1], semI[islot]).wait()
            pltpu.make_async_copy(tgtp_hbm.at[pl.ds(off, CH)],
                                  idxb.at[islot, 2], semI[islot]).wait()

        def issue_g(q, b, islot):
            pltpu.async_copy(tabAc.at[idxb.at[islot, 0]], sbuf[b], semA[b])
            pltpu.async_copy(tabBc.at[idxb.at[islot, 1]], tbuf[b], semB[b])

        def wait_g(b, islot):
            pltpu.make_async_copy(
                tabAc.at[idxb.at[islot, 0]], sbuf[b], semA[b]).wait()
            pltpu.make_async_copy(
                tabBc.at[idxb.at[islot, 1]], tbuf[b], semB[b]).wait()

        def compute(b):
            sb, tb, mb = sbuf[b], tbuf[b], mbuf[b]

            @pl.loop(0, CH)
            def _(i):
                for g in range(HALF // 32):
                    sl = pl.ds(g * 16, 16)
                    a = plsc.bitcast(sb[i, sl], jnp.bfloat16)
                    t = plsc.bitcast(tb[i, sl], jnp.bfloat16)
                    m = jnp.maximum(a + t, jnp.bfloat16(0.0))
                    lo, hi = plsc.unpack(m, format=plsc.PackFormat.INTERLEAVED)
                    mb[i, pl.ds(g * 32, 16)] = lo
                    mb[i, pl.ds(g * 32 + 16, 16)] = hi

        def issue_s(b, islot):
            pltpu.async_copy(mbuf[b], acc.at[idxb.at[islot, 2]], semS[b],
                             add=True)

        def wait_s(b, islot):
            pltpu.make_async_copy(
                mbuf[b], acc.at[idxb.at[islot, 2]], semS[b]).wait()

        def body(q, sub, do_wait_s, do_next, do_issue_i):
            b = sub % 2
            islot = sub % 4
            i2 = (sub + 2) % 4
            wait_g(b, islot)
            if do_wait_s:
                wait_s(b, i2)
            if do_issue_i:
                issue_i(q + sub + 2, i2)
            compute(b)
            issue_s(b, islot)
            if do_next:
                wait_i(q + sub + 2, i2)
                issue_g(q + sub + 2, b, i2)

        for k in range(4):
            issue_i(k, k)
        wait_i(0, 0)
        issue_g(0, 0, 0)
        wait_i(1, 1)
        issue_g(1, 1, 1)
        body(0, 0, False, True, False)
        body(0, 1, False, True, False)
        body(0, 2, True, True, True)
        body(0, 3, True, True, True)

        @pl.loop(4, Q - 4, step=4)
        def _(q):
            for sub in range(4):
                body(q, sub, True, True, True)

        body(Q - 4, 0, True, True, True)
        body(Q - 4, 1, True, True, True)
        body(Q - 4, 2, True, False, False)
        body(Q - 4, 3, True, False, False)
        wait_s(0, 2)
        wait_s(1, 3)

        plsc.subcore_barrier()

        @pl.when(s < 15)
        def _():
            pltpu.sync_copy(acc.at[pl.ds(s * 632, 632)],
                            out_hbm.at[pl.ds(s * 632, 632),
                                       pl.ds(c * HALF, HALF)])

        @pl.when(s == 15)
        def _():
            pltpu.sync_copy(acc.at[pl.ds(15 * 632, N - 15 * 632)],
                            out_hbm.at[pl.ds(15 * 632, N - 15 * 632),
                                       pl.ds(c * HALF, HALF)])

    return edge_kernel(tabAf, tabBf, srcg, tgtg, tgtp)


def kernel(x, adj_list_0, adj_list_1, adj_list_2, adj_list_3, W, b):
    adjs = (adj_list_0, adj_list_1, adj_list_2, adj_list_3)
    z = jnp.zeros((E_PAD - E_PER,), jnp.int32)
    dummy = jnp.full((E_PAD - E_PER,), N, jnp.int32)

    srcg = jnp.concatenate(
        [jnp.concatenate([a[:, 0] + t * N, z]) for t, a in enumerate(adjs)])
    tgtg = jnp.concatenate(
        [jnp.concatenate([a[:, 1] + t * N, z]) for t, a in enumerate(adjs)])
    tgtp = jnp.concatenate(
        [jnp.concatenate([a[:, 1], dummy]) for _, a in enumerate(adjs)])

    b8 = jnp.broadcast_to(b[:, None, :], (T, 8, MSG))

    tabA, tabB = _tables(x, W, b8)
    pA, pB = _pack_tables(tabA.reshape(2 * T * N, HALF),
                          tabB.reshape(2 * T * N, HALF))
    return _edge_stage(pA.reshape(2, T * N, HALF // 2),
                       pB.reshape(2, T * N, HALF // 2),
                       srcg, tgtg, tgtp)


# speedup vs baseline: 1.1010x; 1.1010x over previous
import dataclasses
import functools

import jax
import jax.numpy as jnp
from jax import lax
from jax.experimental import pallas as pl
from jax.experimental.pallas import tpu as pltpu
from jax.experimental.pallas import tpu_sc as plsc

N = 10000
D = 256
MSG = 256
T = 4
E_PER = 40000

NACC = 10112
E_PAD = 40960
CH = 80
CHUNKS = E_PAD // (16 * CH)
HALF = 128
NB = 2000
PB = 250
PROWS = 2 * T * N // 32


def _sc_params():
    cp = pltpu.CompilerParams()
    fields = pltpu.CompilerParams.__dataclass_fields__
    if "needs_layout_passes" in fields:
        cp = dataclasses.replace(cp, needs_layout_passes=False)
    if "use_tc_tiling_on_sc" in fields:
        cp = dataclasses.replace(cp, use_tc_tiling_on_sc=False)
    return cp


def _tables(x, W, b8):
    nblk = N // NB

    def mm(x_ref, w_ref, b_ref, a_ref, t_ref):
        xa = x_ref[...]
        w = w_ref[0]
        a_ref[0, 0] = jnp.dot(xa, w[:D, :], preferred_element_type=jnp.float32)
        t_ref[0, 0] = (jnp.dot(xa, w[D:, :], preferred_element_type=jnp.float32)
                       + b_ref[0, 0][None, :])

    return pl.pallas_call(
        mm,
        grid=(nblk, 2, T),
        in_specs=[
            pl.BlockSpec((NB, D), lambda nb, c, t: (nb, 0)),
            pl.BlockSpec((1, 2 * D, HALF), lambda nb, c, t: (t, 0, c)),
            pl.BlockSpec((1, 8, HALF), lambda nb, c, t: (t, 0, c)),
        ],
        out_specs=[
            pl.BlockSpec((1, 1, NB, HALF), lambda nb, c, t: (c, t, nb, 0)),
            pl.BlockSpec((1, 1, NB, HALF), lambda nb, c, t: (c, t, nb, 0)),
        ],
        out_shape=[jax.ShapeDtypeStruct((2, T, N, HALF), jnp.float32)] * 2,
    )(x, W, b8)


def _pack_tables(tabAf, tabBf):
    mesh = plsc.VectorSubcoreMesh(core_axis_name="c", subcore_axis_name="s")
    rows = 2 * T * N

    nblk = PROWS // PB

    @functools.partial(
        pl.kernel,
        out_type=[jax.ShapeDtypeStruct((rows, HALF // 2), jnp.int32)] * 2,
        mesh=mesh,
        compiler_params=_sc_params(),
        scratch_types=[
            pltpu.VMEM((PB, HALF), jnp.float32),
            pltpu.VMEM((PB, HALF), jnp.float32),
            pltpu.VMEM((PB, HALF // 2), jnp.int32),
            pltpu.VMEM((PB, HALF // 2), jnp.int32),
            pltpu.SemaphoreType.DMA,
            pltpu.SemaphoreType.DMA,
            pltpu.SemaphoreType.DMA,
            pltpu.SemaphoreType.DMA,
        ],
    )
    def pack_kernel(a_hbm, b_hbm, oa_hbm, ob_hbm,
                    f0, f1, o0, o1, semI0, semI1, semO0, semO1):
        c = lax.axis_index("c")
        s = lax.axis_index("s")
        w = s * 2 + c
        base = w * PROWS
        fin, iout = (f0, f1), (o0, o1)
        semI, semO = (semI0, semI1), (semO0, semO1)

        def refs(k):
            src, dst = (a_hbm, oa_hbm) if k < nblk else (b_hbm, ob_hbm)
            r0 = base + (k % nblk) * PB
            return src.at[pl.ds(r0, PB)], dst.at[pl.ds(r0, PB)]

        def issue_in(k, bb):
            pltpu.async_copy(refs(k)[0], fin[bb], semI[bb])

        def wait_in(k, bb):
            pltpu.make_async_copy(refs(k)[0], fin[bb], semI[bb]).wait()

        def issue_out(k, bb):
            pltpu.async_copy(iout[bb], refs(k)[1], semO[bb])

        def wait_out(k, bb):
            pltpu.make_async_copy(iout[bb], refs(k)[1], semO[bb]).wait()

        def compute(bb):
            fi, io = fin[bb], iout[bb]

            @pl.loop(0, PB)
            def _(i):
                for g in range(HALF // 32):
                    lo = fi[i, pl.ds(g * 32, 16)]
                    hi = fi[i, pl.ds(g * 32 + 16, 16)]
                    pk = plsc.pack(lo, hi, format=plsc.PackFormat.INTERLEAVED)
                    io[i, pl.ds(g * 16, 16)] = plsc.bitcast(pk, jnp.int32)

        nb2 = 2 * nblk
        issue_in(0, 0)
        issue_in(1, 1)
        for k in range(nb2):
            bb = k % 2
            wait_in(k, bb)
            if k >= 2:
                wait_out(k - 2, bb)
            compute(bb)
            issue_out(k, bb)
            if k + 2 < nb2:
                issue_in(k + 2, bb)
        wait_out(nb2 - 2, 0)
        wait_out(nb2 - 1, 1)

    return pack_kernel(tabAf, tabBf)


def _edge_stage(tabAf, tabBf, srcg, tgtg, tgtp):
    mesh = plsc.VectorSubcoreMesh(core_axis_name="c", subcore_axis_name="s")
    Q = T * CHUNKS

    @functools.partial(
        pl.kernel,
        out_type=jax.ShapeDtypeStruct((N, 2 * HALF), jnp.float32),
        mesh=mesh,
        compiler_params=_sc_params(),
        scratch_types=[
            pltpu.VMEM((4, 3, CH), jnp.int32),
            pltpu.VMEM((CH, HALF // 2), jnp.int32),
            pltpu.VMEM((CH, HALF // 2), jnp.int32),
            pltpu.VMEM((CH, HALF // 2), jnp.int32),
            pltpu.VMEM((CH, HALF // 2), jnp.int32),
            pltpu.VMEM((CH, HALF), jnp.float32),
            pltpu.VMEM((CH, HALF), jnp.float32),
            pltpu.VMEM_SHARED((NACC, HALF), jnp.float32),
            pltpu.SemaphoreType.DMA,
            pltpu.SemaphoreType.DMA,
            pltpu.SemaphoreType.DMA,
            pltpu.SemaphoreType.DMA,
            pltpu.SemaphoreType.DMA,
            pltpu.SemaphoreType.DMA,
            pltpu.SemaphoreType.DMA,
            pltpu.SemaphoreType.DMA,
            pltpu.SemaphoreType.DMA,
            pltpu.SemaphoreType.DMA,
        ],
    )
    def edge_kernel(tabA_hbm, tabB_hbm, srcg_hbm, tgtg_hbm, tgtp_hbm, out_hbm,
                    idxb, sb0, sb1, tb0, tb1, mb0, mb1, acc,
                    semI0, semI1, semI2, semI3,
                    semA0, semA1, semB0, semB1, semS0, semS1):
        c = lax.axis_index("c")
        s = lax.axis_index("s")
        sbuf, tbuf, mbuf = (sb0, sb1), (tb0, tb1), (mb0, mb1)
        semI = (semI0, semI1, semI2, semI3)
        semA, semB, semS = (semA0, semA1), (semB0, semB1), (semS0, semS1)
        tabAc = tabA_hbm.at[c]
        tabBc = tabB_hbm.at[c]

        @pl.loop(0, CH)
        def _(i):
            for j in range(HALF // 16):
                mb0[i, pl.ds(j * 16, 16)] = jnp.zeros((16,), jnp.float32)

        rows_per_sub = NACC // 16
        nz, rz = rows_per_sub // CH, rows_per_sub % CH

        @pl.loop(0, nz)
        def _(k):
            pltpu.sync_copy(mb0, acc.at[pl.ds(s * rows_per_sub + k * CH, CH)])

        if rz:
            pltpu.sync_copy(mb0.at[pl.ds(0, rz)],
                            acc.at[pl.ds(s * rows_per_sub + nz * CH, rz)])

        plsc.subcore_barrier()

        def _off(q):
            return ((q >> 5) * E_PAD + s * (CHUNKS * CH) + (q & 31) * CH)

        def issue_i(q, islot):
            off = _off(q)
            pltpu.async_copy(srcg_hbm.at[pl.ds(off, CH)], idxb.at[islot, 0],
                             semI[islot])
            pltpu.async_copy(tgtg_hbm.at[pl.ds(off, CH)], idxb.at[islot, 1],
                             semI[islot])
            pltpu.async_copy(tgtp_hbm.at[pl.ds(off, CH)], idxb.at[islot, 2],
                             semI[islot])

        def wait_i(q, islot):
            off = _off(q)
            pltpu.make_async_copy(srcg_hbm.at[pl.ds(off, CH)],
                                  idxb.at[islot, 0], semI[islot]).wait()
            pltpu.make_async_copy(tgtg_hbm.at[pl.ds(off, CH)],
                                  idxb.at[islot, 1], semI[islot]).wait()
            pltpu.make_async_copy(tgtp_hbm.at[pl.ds(off, CH)],
                                  idxb.at[islot, 2], semI[islot]).wait()

        def issue_g(q, b, islot):
            pltpu.async_copy(tabAc.at[idxb.at[islot, 0]], sbuf[b], semA[b])
            pltpu.async_copy(tabBc.at[idxb.at[islot, 1]], tbuf[b], semB[b])

        def wait_g(b, islot):
            pltpu.make_async_copy(
                tabAc.at[idxb.at[islot, 0]], sbuf[b], semA[b]).wait()
            pltpu.make_async_copy(
                tabBc.at[idxb.at[islot, 1]], tbuf[b], semB[b]).wait()

        def compute(b):
            sb, tb, mb = sbuf[b], tbuf[b], mbuf[b]

            @pl.loop(0, CH)
            def _(i):
                for g in range(HALF // 32):
                    sl = pl.ds(g * 16, 16)
                    a = plsc.bitcast(sb[i, sl], jnp.bfloat16)
                    t = plsc.bitcast(tb[i, sl], jnp.bfloat16)
                    m = jnp.maximum(a + t, jnp.bfloat16(0.0))
                    lo, hi = plsc.unpack(m, format=plsc.PackFormat.INTERLEAVED)
                    mb[i, pl.ds(g * 32, 16)] = lo
                    mb[i, pl.ds(g * 32 + 16, 16)] = hi

        def issue_s(b, islot):
            pltpu.async_copy(mbuf[b], acc.at[idxb.at[islot, 2]], semS[b],
                             add=True)

        def wait_s(b, islot):
            pltpu.make_async_copy(
                mbuf[b], acc.at[idxb.at[islot, 2]], semS[b]).wait()

        def body(q, sub, do_wait_s, do_next, do_issue_i):
            b = sub % 2
            islot = sub % 4
            i2 = (sub + 2) % 4
            wait_g(b, islot)
            if do_wait_s:
                wait_s(b, i2)
            if do_issue_i:
                issue_i(q + sub + 2, i2)
            compute(b)
            issue_s(b, islot)
            if do_next:
                wait_i(q + sub + 2, i2)
                issue_g(q + sub + 2, b, i2)

        for k in range(4):
            issue_i(k, k)
        wait_i(0, 0)
        issue_g(0, 0, 0)
        wait_i(1, 1)
        issue_g(1, 1, 1)
        body(0, 0, False, True, False)
        body(0, 1, False, True, False)
        body(0, 2, True, True, True)
        body(0, 3, True, True, True)

        @pl.loop(4, Q - 4, step=4)
        def _(q):
            for sub in range(4):
                body(q, sub, True, True, True)

        body(Q - 4, 0, True, True, True)
        body(Q - 4, 1, True, True, True)
        body(Q - 4, 2, True, False, False)
        body(Q - 4, 3, True, False, False)
        wait_s(0, 2)
        wait_s(1, 3)

        plsc.subcore_barrier()

        @pl.when(s < 15)
        def _():
            pltpu.sync_copy(acc.at[pl.ds(s * 632, 632)],
                            out_hbm.at[pl.ds(s * 632, 632),
                                       pl.ds(c * HALF, HALF)])

        @pl.when(s == 15)
        def _():
            pltpu.sync_copy(acc.at[pl.ds(15 * 632, N - 15 * 632)],
                            out_hbm.at[pl.ds(15 * 632, N - 15 * 632),
                                       pl.ds(c * HALF, HALF)])

    return edge_kernel(tabAf, tabBf, srcg, tgtg, tgtp)


def kernel(x, adj_list_0, adj_list_1, adj_list_2, adj_list_3, W, b):
    adjs = (adj_list_0, adj_list_1, adj_list_2, adj_list_3)
    z = jnp.zeros((E_PAD - E_PER,), jnp.int32)
    dummy = jnp.full((E_PAD - E_PER,), N, jnp.int32)

    srcg = jnp.concatenate(
        [jnp.concatenate([a[:, 0] + t * N, z]) for t, a in enumerate(adjs)])
    tgtg = jnp.concatenate(
        [jnp.concatenate([a[:, 1] + t * N, z]) for t, a in enumerate(adjs)])
    tgtp = jnp.concatenate(
        [jnp.concatenate([a[:, 1], dummy]) for _, a in enumerate(adjs)])

    b8 = jnp.broadcast_to(b[:, None, :], (T, 8, MSG))

    tabA, tabB = _tables(x, W, b8)
    pA, pB = _pack_tables(tabA.reshape(2 * T * N, HALF),
                          tabB.reshape(2 * T * N, HALF))
    return _edge_stage(pA.reshape(2, T * N, HALF // 2),
                       pB.reshape(2, T * N, HALF // 2),
                       srcg, tgtg, tgtp)
